# Initial kernel scaffold; baseline (speedup 1.0000x reference)
import jax
import jax.numpy as jnp
from jax.experimental import pallas as pl


def kernel(first_idx, last_idx, table):
    raise NotImplementedError("write your pallas kernel here")


# speedup vs baseline: 4.9946x; 4.9946x over previous
import functools

import jax
import jax.numpy as jnp
from jax import lax
from jax.experimental import pallas as pl
from jax.experimental.pallas import tpu as pltpu
from jax.experimental.pallas import tpu_sc as plsc

NC, NS = 2, 16
NW = NC * NS
G = 128
KG = 5
CH = KG * G


@functools.lru_cache(maxsize=None)
def _make_gather(n_rows: int, dim: int):
    assert n_rows % (NW * CH) == 0
    rpw = n_rows // NW
    gpw = rpw // G
    nch = rpw // CH

    mesh = plsc.VectorSubcoreMesh(core_axis_name="c", subcore_axis_name="s")

    @functools.partial(
        pl.kernel,
        out_type=jax.ShapeDtypeStruct((n_rows, dim), jnp.float32),
        mesh=mesh,
        compiler_params=pltpu.CompilerParams(use_tc_tiling_on_sc=False),
        scratch_types=[
            pltpu.VMEM((gpw, G), jnp.int32),
            pltpu.VMEM((2, CH, dim), jnp.float32),
            pltpu.SemaphoreType.DMA,
            pltpu.SemaphoreType.DMA,
            pltpu.SemaphoreType.DMA,
            pltpu.SemaphoreType.DMA,
        ],
    )
    def gather_kernel(idx_hbm, table_hbm, out_hbm, idx_v, rows_v,
                      gsem0, gsem1, osem0, osem1):
        wid = lax.axis_index("s") * NC + lax.axis_index("c")
        gsem = (gsem0, gsem1)
        osem = (osem0, osem1)

        pltpu.sync_copy(idx_hbm.at[wid], idx_v)

        def fire_chunk(c, buf):
            descs = []
            for k in range(KG):
                g = c * KG + k
                descs.append(pltpu.async_copy(
                    table_hbm.at[idx_v.at[g]],
                    rows_v.at[buf, pl.ds(k * G, G)],
                    gsem[buf]))
            return descs

        def writeback(c, buf):
            return pltpu.async_copy(
                rows_v.at[buf],
                out_hbm.at[pl.ds(wid * rpw + c * CH, CH)],
                osem[buf])

        out_descs = [None, None]
        gat_descs = fire_chunk(0, 0)
        for c in range(1, nch):
            buf = c & 1
            if out_descs[buf] is not None:
                out_descs[buf].wait()
            new_descs = fire_chunk(c, buf)
            for d in gat_descs:
                d.wait()
            out_descs[1 - buf] = writeback(c - 1, 1 - buf)
            gat_descs = new_descs
        last_buf = (nch - 1) & 1
        for d in gat_descs:
            d.wait()
        out_descs[last_buf] = writeback(nch - 1, last_buf)
        for d in out_descs:
            if d is not None:
                d.wait()

    return gather_kernel


def kernel(first_idx, last_idx, table):
    b, l = first_idx.shape
    dim = table.shape[1]
    n_rows = 2 * b * l
    idx = jnp.stack(
        [first_idx.reshape(-1).astype(jnp.int32),
         last_idx.reshape(-1).astype(jnp.int32)], axis=1).reshape(-1)
    idx = idx.reshape(NW, n_rows // (NW * G), G)
    out = _make_gather(n_rows, dim)(idx, table)
    return out.reshape(b, l, 2 * dim)


# speedup vs baseline: 8.7026x; 1.7424x over previous
import functools

import jax
import jax.numpy as jnp
from jax import lax
from jax.experimental import pallas as pl
from jax.experimental.pallas import tpu as pltpu
from jax.experimental.pallas import tpu_sc as plsc

NC, NS = 2, 16
NW = NC * NS
G = 128
KG = 5
CH = KG * G


@functools.lru_cache(maxsize=None)
def _make_gather(n_rows: int, dim: int):
    assert n_rows % (NW * CH) == 0
    rpw = n_rows // NW
    gpw = rpw // G
    nch = rpw // CH

    mesh = plsc.VectorSubcoreMesh(core_axis_name="c", subcore_axis_name="s")

    @functools.partial(
        pl.kernel,
        out_type=jax.ShapeDtypeStruct((n_rows, dim), jnp.float32),
        mesh=mesh,
        compiler_params=pltpu.CompilerParams(use_tc_tiling_on_sc=False),
        scratch_types=[
            pltpu.VMEM((gpw, G), jnp.int32),
            pltpu.VMEM((2, CH, dim), jnp.float32),
            pltpu.SemaphoreType.DMA,
            pltpu.SemaphoreType.DMA,
            pltpu.SemaphoreType.DMA,
            pltpu.SemaphoreType.DMA,
        ],
    )
    def gather_kernel(idx_hbm, table_hbm, out_hbm, idx_v, rows_v,
                      gsem0, gsem1, osem0, osem1):
        wid = lax.axis_index("s") * NC + lax.axis_index("c")
        gsem = (gsem0, gsem1)
        osem = (osem0, osem1)

        pltpu.sync_copy(idx_hbm.at[wid], idx_v)

        def fire_chunk(c, buf):
            descs = []
            for k in range(KG):
                g = c * KG + k
                descs.append(pltpu.async_copy(
                    table_hbm.at[idx_v.at[g]],
                    rows_v.at[buf, pl.ds(k * G, G)],
                    gsem[buf]))
            return descs

        def writeback(c, buf):
            return pltpu.async_copy(
                rows_v.at[buf],
                out_hbm.at[pl.ds(wid * rpw + c * CH, CH)],
                osem[buf])

        out_descs = [None, None]
        gat_descs = fire_chunk(0, 0)
        for c in range(1, nch):
            buf = c & 1
            if out_descs[buf] is not None:
                out_descs[buf].wait()
            new_descs = fire_chunk(c, buf)
            for d in gat_descs:
                d.wait()
            out_descs[1 - buf] = writeback(c - 1, 1 - buf)
            gat_descs = new_descs
        last_buf = (nch - 1) & 1
        for d in gat_descs:
            d.wait()
        out_descs[last_buf] = writeback(nch - 1, last_buf)
        for d in out_descs:
            if d is not None:
                d.wait()

    return gather_kernel


def kernel(first_idx, last_idx, table):
    b, l = first_idx.shape
    dim = table.shape[1]
    n_rows = 2 * b * l
    idx = jnp.stack(
        [first_idx.T.reshape(-1).astype(jnp.int32),
         last_idx.T.reshape(-1).astype(jnp.int32)], axis=1).reshape(-1)
    idx = idx.reshape(NW, n_rows // (NW * G), G)
    out = _make_gather(n_rows, dim)(idx, table)
    return out.reshape(l, b, 2 * dim).transpose(1, 0, 2)


# speedup vs baseline: 14.6673x; 1.6854x over previous
import functools

import jax
import jax.numpy as jnp
from jax import lax
from jax.experimental import pallas as pl
from jax.experimental.pallas import tpu as pltpu
from jax.experimental.pallas import tpu_sc as plsc

NC, NS = 2, 16
NW = NC * NS
G = 128
KG = 2
CH = KG * G


@functools.lru_cache(maxsize=None)
def _make_gather(n_tok: int, dim: int):
    assert n_tok % (NW * CH) == 0
    tpw = n_tok // NW
    nch = tpw // CH

    mesh = plsc.VectorSubcoreMesh(core_axis_name="c", subcore_axis_name="s")

    @functools.partial(
        pl.kernel,
        out_type=jax.ShapeDtypeStruct((n_tok, 2 * dim), jnp.float32),
        mesh=mesh,
        compiler_params=pltpu.CompilerParams(use_tc_tiling_on_sc=False),
        scratch_types=[
            pltpu.VMEM((tpw,), jnp.int32),
            pltpu.VMEM((tpw,), jnp.int32),
            pltpu.VMEM((2, CH, dim), jnp.float32),
            pltpu.VMEM((2, CH, dim), jnp.float32),
            pltpu.SemaphoreType.DMA,
            pltpu.SemaphoreType.DMA,
            pltpu.SemaphoreType.DMA,
            pltpu.SemaphoreType.DMA,
        ],
    )
    def gather_kernel(fi_hbm, li_hbm, table_hbm, out_hbm, fi_v, li_v,
                      rows_f, rows_l, gsem0, gsem1, osem0, osem1):
        wid = lax.axis_index("s") * NC + lax.axis_index("c")
        base = wid * tpw
        gsem = (gsem0, gsem1)
        osem = (osem0, osem1)

        pltpu.sync_copy(fi_hbm.at[pl.ds(base, tpw)], fi_v)
        pltpu.sync_copy(li_hbm.at[pl.ds(base, tpw)], li_v)

        def fire_chunk(c, buf):
            descs = []
            for k in range(KG):
                g = c * KG + k
                dst_rows = pl.ds(k * G, G)
                descs.append(pltpu.async_copy(
                    table_hbm.at[fi_v.at[pl.ds(g * G, G)]],
                    rows_f.at[buf, dst_rows],
                    gsem[buf]))
                descs.append(pltpu.async_copy(
                    table_hbm.at[li_v.at[pl.ds(g * G, G)]],
                    rows_l.at[buf, dst_rows],
                    gsem[buf]))
            return descs

        def writeback(c, buf):
            rows = pl.ds(base + c * CH, CH)
            d0 = pltpu.async_copy(
                rows_f.at[buf], out_hbm.at[rows, pl.ds(0, dim)], osem[buf])
            d1 = pltpu.async_copy(
                rows_l.at[buf], out_hbm.at[rows, pl.ds(dim, dim)], osem[buf])
            return (d0, d1)

        out_descs = [None, None]
        gat_descs = fire_chunk(0, 0)
        for c in range(1, nch):
            buf = c & 1
            if out_descs[buf] is not None:
                for d in out_descs[buf]:
                    d.wait()
            new_descs = fire_chunk(c, buf)
            for d in gat_descs:
                d.wait()
            out_descs[1 - buf] = writeback(c - 1, 1 - buf)
            gat_descs = new_descs
        last_buf = (nch - 1) & 1
        for d in gat_descs:
            d.wait()
        out_descs[last_buf] = writeback(nch - 1, last_buf)
        for ds2 in out_descs:
            if ds2 is not None:
                for d in ds2:
                    d.wait()

    return gather_kernel


def kernel(first_idx, last_idx, table):
    b, l = first_idx.shape
    dim = table.shape[1]
    n_tok = b * l
    fi = first_idx.T.reshape(-1).astype(jnp.int32)
    li = last_idx.T.reshape(-1).astype(jnp.int32)
    out = _make_gather(n_tok, dim)(fi, li, table)
    return out.reshape(l, b, 2 * dim).transpose(1, 0, 2)
